# Initial kernel scaffold; baseline (speedup 1.0000x reference)
import jax
import jax.numpy as jnp
from jax.experimental import pallas as pl


def kernel(embedding, entity_embedding, entity_mask, key_fc_w, key_fc_b, qm1_w, qm1_b, qm2_w, qm2_b, em1_w, em1_b, em2_w, em2_b, lstm_Wih, lstm_Whh, lstm_g_ih, lstm_b_ih, lstm_g_hh, lstm_b_hh, lstm_g_c, lstm_b_c, enc_w, enc_b, actor_w, actor_b, critic_w, critic_b):
    raise NotImplementedError("write your pallas kernel here")


# speedup vs baseline: 1.2229x; 1.2229x over previous
import jax
import jax.numpy as jnp
from jax.experimental import pallas as pl
from jax.experimental.pallas import tpu as pltpu

_BS = 64
_N = 512
_ENT = 256
_KD = 64
_INP = 1024
_FUNC = 256
_H = 64
_L = 3
_STEPS = 64


def _key_proj_kernel(ee_ref, w_ref, b_ref, out_ref):
    out_ref[...] = jax.lax.dot_general(
        ee_ref[...], w_ref[...], (((2,), (1,)), ((), ())),
        preferred_element_type=jnp.float32,
    ) + b_ref[...]


def _r256(x):
    y = x[:, :128] + x[:, 128:]
    acc = y[:, 0:8]
    for j in range(1, 16):
        acc = acc + y[:, 8 * j:8 * j + 8]
    acc = acc[:, :4] + acc[:, 4:]
    acc = acc[:, :2] + acc[:, 2:]
    return acc[:, :1] + acc[:, 1:]


def _r64(x):
    acc = x[:, 0:8]
    for j in range(1, 8):
        acc = acc + x[:, 8 * j:8 * j + 8]
    acc = acc[:, :4] + acc[:, 4:]
    acc = acc[:, :2] + acc[:, 2:]
    return acc[:, :1] + acc[:, 1:]


def _ln256(x, g, b):
    m = _r256(x) * (1.0 / 256.0)
    d = x - m
    v = _r256(d * d) * (1.0 / 256.0)
    return d / jnp.sqrt(v + 1e-5) * g + b


def _ln64(x, g, b):
    m = _r64(x) * (1.0 / 64.0)
    d = x - m
    v = _r64(d * d) * (1.0 / 64.0)
    return d / jnp.sqrt(v + 1e-5) * g + b


def _sigmoid(x):
    return 1.0 / (jnp.exp(-x) + 1.0)


def _dotT(a, w):
    return jax.lax.dot_general(a, w, (((1,), (1,)), ((), ())),
                               preferred_element_type=jnp.float32)


def _attn(x, key_bkn):
    acc = x[:, 0:8, None] * key_bkn[:, 0:8, :]
    for j in range(1, 8):
        acc = acc + x[:, 8 * j:8 * j + 8, None] * key_bkn[:, 8 * j:8 * j + 8, :]
    acc = acc[:, :4, :] + acc[:, 4:, :]
    acc = acc[:, :2, :] + acc[:, 2:, :]
    acc = acc[:, :1, :] + acc[:, 1:, :]
    return acc[:, 0, :]


def _sel_sum(key_bkn, oh):
    t = key_bkn[..., 0:128] * oh[:, None, 0:128]
    for g in range(1, 4):
        t = t + key_bkn[..., 128 * g:128 * (g + 1)] * oh[:, None,
                                                         128 * g:128 * (g + 1)]
    a = t + jnp.roll(t, -1, axis=-1)
    bb = a + jnp.roll(a, -2, axis=-1)
    c = bb + jnp.roll(bb, -4, axis=-1)
    acc = c[..., 0:1]
    for g in range(1, 16):
        acc = acc + c[..., 8 * g:8 * g + 1]
    return acc[..., 0]


def _main_kernel(
    emb_ref, key_ref, mask_ref, gum_ref,
    qm1_ref, qm1b_ref, qm2_ref, qm2b_ref,
    Wih_ref, gih_ref, bih_ref, Whh_ref, ghh_ref, bhh_ref,
    gc_ref, bc_ref,
    em1_ref, em1b_ref, em2_ref, em2b_ref,
    enc_ref, encb_ref, actw_ref, actb_ref, criw_ref, crib_ref,
    logits_ref, res_ref, ae_ref, vl_ref, val_ref,
    gum_scr, log_scr, gum_sem, log_sem,
):
    emb = emb_ref[...]
    key_bkn = key_ref[...]
    iota_n = jax.lax.broadcasted_iota(jnp.int32, (_BS, _N), 1)
    iota_s = jax.lax.broadcasted_iota(jnp.int32, (_BS, _STEPS), 1)

    def body(step, carry):
        mask, oh, ae, res, h0, h1, h2, c0, c1, c2 = carry
        cp = pltpu.make_async_copy(gum_ref.at[step], gum_scr, gum_sem)
        cp.start()
        x = jnp.maximum(_dotT(ae, qm1_ref[...]) + qm1b_ref[...], 0.0)
        x = _dotT(x, qm2_ref[...]) + qm2b_ref[...]
        hs = [h0, h1, h2]
        cs = [c0, c1, c2]
        for l in range(_L):
            ig = _ln256(_dotT(x, Wih_ref[l]), gih_ref[l], bih_ref[l])
            hg = _ln256(_dotT(hs[l], Whh_ref[l]), ghh_ref[l], bhh_ref[l])
            s = ig + hg
            gi = _sigmoid(s[:, 0 * _H:1 * _H])
            gf = _sigmoid(s[:, 1 * _H:2 * _H])
            gg = jnp.tanh(s[:, 2 * _H:3 * _H])
            go = _sigmoid(s[:, 3 * _H:4 * _H])
            c = _ln64(gf * cs[l] + gi * gg, gc_ref[l], bc_ref[l])
            h = go * jnp.tanh(c)
            hs[l] = h
            cs[l] = c
            x = h
        logits = _attn(x, key_bkn)
        masked = jnp.where(mask > 0, logits, -1.0e9)
        log_scr[...] = masked
        lcp = pltpu.make_async_copy(log_scr, logits_ref.at[step], log_sem)
        lcp.start()
        cp.wait()
        noisy = masked + gum_scr[...]
        lcp.wait()
        result = jnp.argmax(noisy, axis=-1).astype(jnp.int32)
        res = jnp.where(iota_s == step, result[:, None], res)
        hit = iota_n == result[:, None]
        mask = jnp.where(hit, 0.0, mask)
        oh = jnp.where(hit, 1.0, oh)
        sel = _sel_sum(key_bkn, oh) / (step + 1).astype(jnp.float32)
        e = jnp.maximum(_dotT(sel, em1_ref[...]) + em1b_ref[...], 0.0)
        ae = emb + (_dotT(e, em2_ref[...]) + em2b_ref[...])
        return (mask, oh, ae, res, *hs, *cs)

    z = jnp.zeros((_BS, _H), jnp.float32)
    carry = (
        mask_ref[...],
        jnp.zeros((_BS, _N), jnp.float32),
        emb,
        jnp.zeros((_BS, _STEPS), jnp.int32),
        z, z, z, z, z, z,
    )
    carry = jax.lax.fori_loop(0, _STEPS, body, carry)
    ae = carry[2]
    res_ref[...] = carry[3]
    ae_ref[...] = ae
    enc = jnp.maximum(_dotT(ae, enc_ref[...]) + encb_ref[...], 0.0)
    vl_ref[...] = jnp.sum(enc * actw_ref[...], axis=-1, keepdims=True) \
        + actb_ref[...]
    val_ref[...] = jnp.sum(enc * criw_ref[...], axis=-1, keepdims=True) \
        + crib_ref[...]


def kernel(embedding, entity_embedding, entity_mask, key_fc_w, key_fc_b,
           qm1_w, qm1_b, qm2_w, qm2_b, em1_w, em1_b, em2_w, em2_b,
           lstm_Wih, lstm_Whh, lstm_g_ih, lstm_b_ih, lstm_g_hh, lstm_b_hh,
           lstm_g_c, lstm_b_c, enc_w, enc_b, actor_w, actor_b,
           critic_w, critic_b):
    f32 = jnp.float32
    nblk = 8
    key_t = pl.pallas_call(
        _key_proj_kernel,
        grid=(nblk,),
        in_specs=[
            pl.BlockSpec((_BS // nblk, _N, _ENT), lambda i: (i, 0, 0)),
            pl.BlockSpec((_KD, _ENT), lambda i: (0, 0)),
            pl.BlockSpec((1, _KD), lambda i: (0, 0)),
        ],
        out_specs=pl.BlockSpec((_BS // nblk, _N, _KD), lambda i: (i, 0, 0)),
        out_shape=jax.ShapeDtypeStruct((_BS, _N, _KD), f32),
    )(entity_embedding, key_fc_w, key_fc_b.reshape(1, _KD))
    key_bkn = key_t.transpose(0, 2, 1)

    rkey = jax.random.key(42)
    gum = jax.vmap(
        lambda s: jax.random.gumbel(
            jax.random.fold_in(rkey, s), (_BS, _N), f32
        )
    )(jnp.arange(_STEPS))

    outs = pl.pallas_call(
        _main_kernel,
        in_specs=[pl.BlockSpec(memory_space=pl.ANY)
                  if i == 3 else pl.BlockSpec()
                  for i in range(26)],
        scratch_shapes=[
            pltpu.VMEM((_BS, _N), f32),
            pltpu.VMEM((_BS, _N), f32),
            pltpu.SemaphoreType.DMA,
            pltpu.SemaphoreType.DMA,
        ],
        out_specs=[
            pl.BlockSpec(memory_space=pl.ANY),
            pl.BlockSpec(), pl.BlockSpec(), pl.BlockSpec(), pl.BlockSpec(),
        ],
        out_shape=[
            jax.ShapeDtypeStruct((_STEPS, _BS, _N), f32),
            jax.ShapeDtypeStruct((_BS, _STEPS), jnp.int32),
            jax.ShapeDtypeStruct((_BS, _INP), f32),
            jax.ShapeDtypeStruct((_BS, 1), f32),
            jax.ShapeDtypeStruct((_BS, 1), f32),
        ],
    )(
        embedding, key_bkn, entity_mask.astype(f32), gum,
        qm1_w, qm1_b.reshape(1, _FUNC), qm2_w, qm2_b.reshape(1, _KD),
        lstm_Wih, lstm_g_ih.reshape(_L, 1, 4 * _H),
        lstm_b_ih.reshape(_L, 1, 4 * _H),
        lstm_Whh, lstm_g_hh.reshape(_L, 1, 4 * _H),
        lstm_b_hh.reshape(_L, 1, 4 * _H),
        lstm_g_c.reshape(_L, 1, _H), lstm_b_c.reshape(_L, 1, _H),
        em1_w, em1_b.reshape(1, _FUNC), em2_w, em2_b.reshape(1, _INP),
        enc_w, enc_b.reshape(1, _FUNC),
        actor_w, actor_b.reshape(1, 1), critic_w, critic_b.reshape(1, 1),
    )
    logits, results, ae, value_logit, value = outs
    return (
        logits.transpose(1, 0, 2),
        results,
        ae,
        value_logit,
        value[:, 0],
    )


# speedup vs baseline: 1.2247x; 1.0015x over previous
import jax
import jax.numpy as jnp
from jax.experimental import pallas as pl
from jax.experimental.pallas import tpu as pltpu

_BS = 64
_N = 512
_ENT = 256
_KD = 64
_INP = 1024
_FUNC = 256
_H = 64
_L = 3
_STEPS = 64


def _key_proj_kernel(ee_ref, w_ref, b_ref, out_ref):
    out_ref[...] = jax.lax.dot_general(
        ee_ref[...], w_ref[...], (((2,), (1,)), ((), ())),
        preferred_element_type=jnp.float32,
    ) + b_ref[...]


def _r256(x):
    y = x[:, :128] + x[:, 128:]
    acc = y[:, 0:8]
    for j in range(1, 16):
        acc = acc + y[:, 8 * j:8 * j + 8]
    acc = acc[:, :4] + acc[:, 4:]
    acc = acc[:, :2] + acc[:, 2:]
    return acc[:, :1] + acc[:, 1:]


def _r64(x):
    acc = x[:, 0:8]
    for j in range(1, 8):
        acc = acc + x[:, 8 * j:8 * j + 8]
    acc = acc[:, :4] + acc[:, 4:]
    acc = acc[:, :2] + acc[:, 2:]
    return acc[:, :1] + acc[:, 1:]


def _ln256(x, g, b):
    m = _r256(x) * (1.0 / 256.0)
    d = x - m
    v = _r256(d * d) * (1.0 / 256.0)
    return d / jnp.sqrt(v + 1e-5) * g + b


def _ln64(x, g, b):
    m = _r64(x) * (1.0 / 64.0)
    d = x - m
    v = _r64(d * d) * (1.0 / 64.0)
    return d / jnp.sqrt(v + 1e-5) * g + b


def _sigmoid(x):
    return 1.0 / (jnp.exp(-x) + 1.0)


def _dotT(a, w):
    return jax.lax.dot_general(a, w, (((1,), (1,)), ((), ())),
                               preferred_element_type=jnp.float32)


def _bfrt(x):
    return x.astype(jnp.bfloat16).astype(jnp.float32)


def _attn(x, key_bkn):
    acc = x[:, 0:8, None] * key_bkn[:, 0:8, :]
    for j in range(1, 8):
        acc = acc + x[:, 8 * j:8 * j + 8, None] * key_bkn[:, 8 * j:8 * j + 8, :]
    acc = acc[:, :4, :] + acc[:, 4:, :]
    acc = acc[:, :2, :] + acc[:, 2:, :]
    acc = acc[:, :1, :] + acc[:, 1:, :]
    return acc[:, 0, :]


def _sel_sum(key_bkn, oh):
    t = key_bkn[..., 0:128] * oh[:, None, 0:128]
    for g in range(1, 4):
        t = t + key_bkn[..., 128 * g:128 * (g + 1)] * oh[:, None,
                                                         128 * g:128 * (g + 1)]
    a = t + jnp.roll(t, -1, axis=-1)
    bb = a + jnp.roll(a, -2, axis=-1)
    c = bb + jnp.roll(bb, -4, axis=-1)
    acc = c[..., 0:1]
    for g in range(1, 16):
        acc = acc + c[..., 8 * g:8 * g + 1]
    return acc[..., 0]


def _main_kernel(
    emb_ref, key_ref, mask_ref, gum_ref,
    qm1_ref, qm1b_ref, qm2_ref, qm2b_ref,
    Wih_ref, gih_ref, bih_ref, Whh_ref, ghh_ref, bhh_ref,
    gc_ref, bc_ref,
    em1_ref, em1b_ref, em2_ref, em2b_ref,
    enc_ref, encb_ref, actw_ref, actb_ref, criw_ref, crib_ref,
    logits_ref, res_ref, ae_ref, vl_ref, val_ref,
    gum_scr, log_scr, gum_sem, log_sem,
):
    emb = emb_ref[...]
    key_bkn = key_ref[...]
    iota_n = jax.lax.broadcasted_iota(jnp.int32, (_BS, _N), 1)
    iota_s = jax.lax.broadcasted_iota(jnp.int32, (_BS, _STEPS), 1)

    def body(step, carry):
        mask, oh, ae, res, h0, h1, h2, c0, c1, c2 = carry
        cp = pltpu.make_async_copy(gum_ref.at[step], gum_scr, gum_sem)
        cp.start()
        x = jnp.maximum(_dotT(ae, qm1_ref[...]) + qm1b_ref[...], 0.0)
        x = _dotT(_bfrt(x), qm2_ref[...]) + qm2b_ref[...]
        hs = [h0, h1, h2]
        cs = [c0, c1, c2]
        for l in range(_L):
            ig = _ln256(_dotT(_bfrt(x), _bfrt(Wih_ref[l])),
                        gih_ref[l], bih_ref[l])
            hg = _ln256(_dotT(_bfrt(hs[l]), _bfrt(Whh_ref[l])),
                        ghh_ref[l], bhh_ref[l])
            s = ig + hg
            gi = _sigmoid(s[:, 0 * _H:1 * _H])
            gf = _sigmoid(s[:, 1 * _H:2 * _H])
            gg = jnp.tanh(s[:, 2 * _H:3 * _H])
            go = _sigmoid(s[:, 3 * _H:4 * _H])
            c = _ln64(gf * cs[l] + gi * gg, gc_ref[l], bc_ref[l])
            h = go * jnp.tanh(c)
            hs[l] = h
            cs[l] = c
            x = h
        logits = _attn(x, key_bkn)
        masked = jnp.where(mask > 0, logits, -1.0e9)
        log_scr[...] = masked
        lcp = pltpu.make_async_copy(log_scr, logits_ref.at[step], log_sem)
        lcp.start()
        cp.wait()
        noisy = masked + gum_scr[...]
        lcp.wait()
        result = jnp.argmax(noisy, axis=-1).astype(jnp.int32)
        res = jnp.where(iota_s == step, result[:, None], res)
        hit = iota_n == result[:, None]
        mask = jnp.where(hit, 0.0, mask)
        oh = jnp.where(hit, 1.0, oh)
        sel = _sel_sum(key_bkn, oh) / (step + 1).astype(jnp.float32)
        e = jnp.maximum(_dotT(sel, em1_ref[...]) + em1b_ref[...], 0.0)
        ae = emb + (_dotT(e, em2_ref[...]) + em2b_ref[...])
        return (mask, oh, ae, res, *hs, *cs)

    z = jnp.zeros((_BS, _H), jnp.float32)
    carry = (
        mask_ref[...],
        jnp.zeros((_BS, _N), jnp.float32),
        emb,
        jnp.zeros((_BS, _STEPS), jnp.int32),
        z, z, z, z, z, z,
    )
    carry = jax.lax.fori_loop(0, _STEPS, body, carry)
    ae = carry[2]
    res_ref[...] = carry[3]
    ae_ref[...] = ae
    enc = jnp.maximum(_dotT(ae, enc_ref[...]) + encb_ref[...], 0.0)
    vl_ref[...] = jnp.sum(enc * actw_ref[...], axis=-1, keepdims=True) \
        + actb_ref[...]
    val_ref[...] = jnp.sum(enc * criw_ref[...], axis=-1, keepdims=True) \
        + crib_ref[...]


def kernel(embedding, entity_embedding, entity_mask, key_fc_w, key_fc_b,
           qm1_w, qm1_b, qm2_w, qm2_b, em1_w, em1_b, em2_w, em2_b,
           lstm_Wih, lstm_Whh, lstm_g_ih, lstm_b_ih, lstm_g_hh, lstm_b_hh,
           lstm_g_c, lstm_b_c, enc_w, enc_b, actor_w, actor_b,
           critic_w, critic_b):
    f32 = jnp.float32
    nblk = 8
    key_t = pl.pallas_call(
        _key_proj_kernel,
        grid=(nblk,),
        in_specs=[
            pl.BlockSpec((_BS // nblk, _N, _ENT), lambda i: (i, 0, 0)),
            pl.BlockSpec((_KD, _ENT), lambda i: (0, 0)),
            pl.BlockSpec((1, _KD), lambda i: (0, 0)),
        ],
        out_specs=pl.BlockSpec((_BS // nblk, _N, _KD), lambda i: (i, 0, 0)),
        out_shape=jax.ShapeDtypeStruct((_BS, _N, _KD), f32),
    )(entity_embedding, key_fc_w, key_fc_b.reshape(1, _KD))
    key_bkn = key_t.transpose(0, 2, 1)

    rkey = jax.random.key(42)
    gum = jax.vmap(
        lambda s: jax.random.gumbel(
            jax.random.fold_in(rkey, s), (_BS, _N), f32
        )
    )(jnp.arange(_STEPS))

    outs = pl.pallas_call(
        _main_kernel,
        in_specs=[pl.BlockSpec(memory_space=pl.ANY)
                  if i == 3 else pl.BlockSpec()
                  for i in range(26)],
        scratch_shapes=[
            pltpu.VMEM((_BS, _N), f32),
            pltpu.VMEM((_BS, _N), f32),
            pltpu.SemaphoreType.DMA,
            pltpu.SemaphoreType.DMA,
        ],
        out_specs=[
            pl.BlockSpec(memory_space=pl.ANY),
            pl.BlockSpec(), pl.BlockSpec(), pl.BlockSpec(), pl.BlockSpec(),
        ],
        out_shape=[
            jax.ShapeDtypeStruct((_STEPS, _BS, _N), f32),
            jax.ShapeDtypeStruct((_BS, _STEPS), jnp.int32),
            jax.ShapeDtypeStruct((_BS, _INP), f32),
            jax.ShapeDtypeStruct((_BS, 1), f32),
            jax.ShapeDtypeStruct((_BS, 1), f32),
        ],
    )(
        embedding, key_bkn, entity_mask.astype(f32), gum,
        qm1_w, qm1_b.reshape(1, _FUNC), qm2_w, qm2_b.reshape(1, _KD),
        lstm_Wih, lstm_g_ih.reshape(_L, 1, 4 * _H),
        lstm_b_ih.reshape(_L, 1, 4 * _H),
        lstm_Whh, lstm_g_hh.reshape(_L, 1, 4 * _H),
        lstm_b_hh.reshape(_L, 1, 4 * _H),
        lstm_g_c.reshape(_L, 1, _H), lstm_b_c.reshape(_L, 1, _H),
        em1_w, em1_b.reshape(1, _FUNC), em2_w, em2_b.reshape(1, _INP),
        enc_w, enc_b.reshape(1, _FUNC),
        actor_w, actor_b.reshape(1, 1), critic_w, critic_b.reshape(1, 1),
    )
    logits, results, ae, value_logit, value = outs
    return (
        logits.transpose(1, 0, 2),
        results,
        ae,
        value_logit,
        value[:, 0],
    )
